# baseline (speedup 1.0000x reference)
import functools

import jax
import jax.numpy as jnp
from jax import lax
from jax.experimental import pallas as pl
from jax.experimental.pallas import tpu as pltpu
from jax.experimental.pallas import tpu_sc as plsc

BATCH = 16384
EMB_K = 64
PACKED = 2 * EMB_K
CHUNK = 128
PACK_BLK = 2048


def _pack_body(wt_ref, ht_ref, eye_ref, p_ref):
  dims = (((0,), (0,)), ((), ()))
  eye = eye_ref[...]
  wt_t = jax.lax.dot_general(wt_ref[...], eye, dims,
                             preferred_element_type=jnp.float32)
  ht_t = jax.lax.dot_general(ht_ref[...], eye, dims,
                             preferred_element_type=jnp.float32)
  p_ref[...] = jnp.concatenate([wt_t, ht_t], axis=1)


def _pack_call(WT, HT):
  n = WT.shape[1]
  eye = jnp.eye(EMB_K, dtype=jnp.float32)
  grid = (pl.cdiv(n, PACK_BLK),)
  return pl.pallas_call(
      _pack_body,
      grid=grid,
      in_specs=[
          pl.BlockSpec((EMB_K, PACK_BLK), lambda i: (0, i)),
          pl.BlockSpec((EMB_K, PACK_BLK), lambda i: (0, i)),
          pl.BlockSpec((EMB_K, EMB_K), lambda i: (0, 0)),
      ],
      out_specs=pl.BlockSpec((PACK_BLK, PACKED), lambda i: (i, 0)),
      out_shape=jax.ShapeDtypeStruct((n, PACKED), jnp.float32),
  )(WT, HT, eye)


def _gather_call(uidx3d, vidx3d, P):
  info = plsc.get_sparse_core_info()
  nc, ns = info.num_cores, info.num_subcores
  nw = nc * ns
  rows_per_w = BATCH // nw
  chunks_per_w = rows_per_w // CHUNK

  mesh = plsc.VectorSubcoreMesh(core_axis_name="c", subcore_axis_name="s")

  @functools.partial(
      pl.kernel,
      mesh=mesh,
      out_type=[
          jax.ShapeDtypeStruct((BATCH, PACKED), jnp.float32),
          jax.ShapeDtypeStruct((BATCH, PACKED), jnp.float32),
      ],
      scratch_types=[
          pltpu.VMEM((chunks_per_w, 1, CHUNK), jnp.int32),
          pltpu.VMEM((chunks_per_w, 1, CHUNK), jnp.int32),
          pltpu.VMEM((rows_per_w, PACKED), jnp.float32),
          pltpu.SemaphoreType.DMA,
      ],
  )
  def gather_k(uidx_hbm, vidx_hbm, p_hbm, u_out, v_out,
               uidx_v, vidx_v, rows, sem):
    wid = lax.axis_index("s") * nc + lax.axis_index("c")
    idx_base = wid * chunks_per_w
    row_base = wid * rows_per_w
    pltpu.sync_copy(uidx_hbm.at[pl.ds(idx_base, chunks_per_w)], uidx_v)
    pltpu.sync_copy(vidx_hbm.at[pl.ds(idx_base, chunks_per_w)], vidx_v)
    for idx_v, out in ((uidx_v, u_out), (vidx_v, v_out)):
      cps = []
      for c in range(chunks_per_w):
        cps.append(pltpu.async_copy(
            p_hbm.at[idx_v.at[c, 0]], rows.at[pl.ds(c * CHUNK, CHUNK)], sem))
      for cp in cps:
        cp.wait()
      pltpu.sync_copy(rows, out.at[pl.ds(row_base, rows_per_w)])

  return gather_k(uidx3d, vidx3d, P)


def _mlp_body(u_ref, v_ref, p1u_ref, p1v_ref, b1_ref, w2_ref, b2_ref,
              w3_ref, b3_ref, out_ref):
  h = jnp.dot(u_ref[...], p1u_ref[...], preferred_element_type=jnp.float32)
  h += jnp.dot(v_ref[...], p1v_ref[...], preferred_element_type=jnp.float32)
  h = jnp.maximum(h + b1_ref[...], 0.0)
  h = jnp.dot(h, w2_ref[...], preferred_element_type=jnp.float32)
  h = jnp.maximum(h + b2_ref[...], 0.0)
  out_ref[...] = jnp.sum(h * w3_ref[...], axis=1) + b3_ref[0]


def _mlp_call(U128, V128, P1u, P1v, b1, W2T, b2, w3, b3):
  blk = 2048
  grid = (BATCH // blk,)
  full = lambda shape: pl.BlockSpec(shape, lambda i: (0,) * len(shape))
  return pl.pallas_call(
      _mlp_body,
      grid=grid,
      in_specs=[
          pl.BlockSpec((blk, PACKED), lambda i: (i, 0)),
          pl.BlockSpec((blk, PACKED), lambda i: (i, 0)),
          full((PACKED, EMB_K)),
          full((PACKED, EMB_K)),
          full((1, EMB_K)),
          full((EMB_K, EMB_K)),
          full((1, EMB_K)),
          full((1, EMB_K)),
          full((1,)),
      ],
      out_specs=pl.BlockSpec((blk,), lambda i: (i,)),
      out_shape=jax.ShapeDtypeStruct((BATCH,), jnp.float32),
  )(U128, V128, P1u, P1v, b1, W2T, b2, w3, b3)


@jax.jit
def kernel(x, W, H, W1, b1, W2, b2, W3, b3):
  uidx = x[:, 0].astype(jnp.int32).reshape(BATCH // CHUNK, 1, CHUNK)
  vidx = x[:, 1].astype(jnp.int32).reshape(BATCH // CHUNK, 1, CHUNK)
  P = _pack_call(W.T, H.T)
  U128, V128 = _gather_call(uidx, vidx, P)
  w1t = W1.T
  zeros = jnp.zeros((EMB_K, EMB_K), jnp.float32)
  P1u = jnp.concatenate([w1t[:EMB_K], zeros], axis=0)
  P1v = jnp.concatenate([zeros, w1t[EMB_K:]], axis=0)
  out = _mlp_call(
      U128, V128, P1u, P1v, b1.reshape(1, EMB_K),
      W2.T, b2.reshape(1, EMB_K),
      W3.reshape(1, EMB_K), b3,
  )
  return out


# speedup vs baseline: 1.4131x; 1.4131x over previous
import functools

import jax
import jax.numpy as jnp
from jax import lax
from jax.experimental import pallas as pl
from jax.experimental.pallas import tpu as pltpu
from jax.experimental.pallas import tpu_sc as plsc

BATCH = 16384
EMB_K = 64
PACKED = 2 * EMB_K
CHUNK = 128
PACK_BLK = 8192


def _pack_body(wt_ref, ht_ref, eye_ref, p_ref):
  dims = (((0,), (0,)), ((), ()))
  eye = eye_ref[...]
  wt_t = jax.lax.dot_general(wt_ref[...], eye, dims,
                             preferred_element_type=jnp.float32)
  ht_t = jax.lax.dot_general(ht_ref[...], eye, dims,
                             preferred_element_type=jnp.float32)
  p_ref[...] = jnp.concatenate([wt_t, ht_t], axis=1)


def _pack_call(WT, HT):
  n = WT.shape[1]
  eye = jnp.eye(EMB_K, dtype=jnp.float32)
  grid = (pl.cdiv(n, PACK_BLK),)
  return pl.pallas_call(
      _pack_body,
      grid=grid,
      in_specs=[
          pl.BlockSpec((EMB_K, PACK_BLK), lambda i: (0, i)),
          pl.BlockSpec((EMB_K, PACK_BLK), lambda i: (0, i)),
          pl.BlockSpec((EMB_K, EMB_K), lambda i: (0, 0)),
      ],
      out_specs=pl.BlockSpec((PACK_BLK, PACKED), lambda i: (i, 0)),
      out_shape=jax.ShapeDtypeStruct((n, PACKED), jnp.float32),
  )(WT, HT, eye)


def _gather_call(uidx3d, vidx3d, P):
  info = plsc.get_sparse_core_info()
  nc, ns = info.num_cores, info.num_subcores
  nw = nc * ns
  rows_per_w = BATCH // nw
  chunks_per_w = rows_per_w // CHUNK

  mesh = plsc.VectorSubcoreMesh(core_axis_name="c", subcore_axis_name="s")

  @functools.partial(
      pl.kernel,
      mesh=mesh,
      out_type=[
          jax.ShapeDtypeStruct((BATCH, PACKED), jnp.float32),
          jax.ShapeDtypeStruct((BATCH, PACKED), jnp.float32),
      ],
      scratch_types=[
          pltpu.VMEM((chunks_per_w, 1, CHUNK), jnp.int32),
          pltpu.VMEM((chunks_per_w, 1, CHUNK), jnp.int32),
          pltpu.VMEM((rows_per_w, PACKED), jnp.float32),
          pltpu.SemaphoreType.DMA,
      ],
  )
  def gather_k(uidx_hbm, vidx_hbm, p_hbm, u_out, v_out,
               uidx_v, vidx_v, rows, sem):
    wid = lax.axis_index("s") * nc + lax.axis_index("c")
    idx_base = wid * chunks_per_w
    row_base = wid * rows_per_w
    pltpu.sync_copy(uidx_hbm.at[pl.ds(idx_base, chunks_per_w)], uidx_v)
    pltpu.sync_copy(vidx_hbm.at[pl.ds(idx_base, chunks_per_w)], vidx_v)
    for idx_v, out in ((uidx_v, u_out), (vidx_v, v_out)):
      cps = []
      for c in range(chunks_per_w):
        cps.append(pltpu.async_copy(
            p_hbm.at[idx_v.at[c, 0]], rows.at[pl.ds(c * CHUNK, CHUNK)], sem))
      for cp in cps:
        cp.wait()
      pltpu.sync_copy(rows, out.at[pl.ds(row_base, rows_per_w)])

  return gather_k(uidx3d, vidx3d, P)


def _mlp_body(u_ref, v_ref, p1u_ref, p1v_ref, b1_ref, w2_ref, b2_ref,
              w3_ref, b3_ref, out_ref):
  h = jnp.dot(u_ref[...], p1u_ref[...], preferred_element_type=jnp.float32)
  h += jnp.dot(v_ref[...], p1v_ref[...], preferred_element_type=jnp.float32)
  h = jnp.maximum(h + b1_ref[...], 0.0)
  h = jnp.dot(h, w2_ref[...], preferred_element_type=jnp.float32)
  h = jnp.maximum(h + b2_ref[...], 0.0)
  out_ref[...] = jnp.sum(h * w3_ref[...], axis=1) + b3_ref[0]


def _mlp_call(U128, V128, P1u, P1v, b1, W2T, b2, w3, b3):
  blk = 2048
  grid = (BATCH // blk,)
  full = lambda shape: pl.BlockSpec(shape, lambda i: (0,) * len(shape))
  return pl.pallas_call(
      _mlp_body,
      grid=grid,
      in_specs=[
          pl.BlockSpec((blk, PACKED), lambda i: (i, 0)),
          pl.BlockSpec((blk, PACKED), lambda i: (i, 0)),
          full((PACKED, EMB_K)),
          full((PACKED, EMB_K)),
          full((1, EMB_K)),
          full((EMB_K, EMB_K)),
          full((1, EMB_K)),
          full((1, EMB_K)),
          full((1,)),
      ],
      out_specs=pl.BlockSpec((blk,), lambda i: (i,)),
      out_shape=jax.ShapeDtypeStruct((BATCH,), jnp.float32),
  )(U128, V128, P1u, P1v, b1, W2T, b2, w3, b3)


@jax.jit
def kernel(x, W, H, W1, b1, W2, b2, W3, b3):
  uidx = x[:, 0].astype(jnp.int32).reshape(BATCH // CHUNK, 1, CHUNK)
  vidx = x[:, 1].astype(jnp.int32).reshape(BATCH // CHUNK, 1, CHUNK)
  P = _pack_call(W.T, H.T)
  U128, V128 = _gather_call(uidx, vidx, P)
  w1t = W1.T
  zeros = jnp.zeros((EMB_K, EMB_K), jnp.float32)
  P1u = jnp.concatenate([w1t[:EMB_K], zeros], axis=0)
  P1v = jnp.concatenate([zeros, w1t[EMB_K:]], axis=0)
  out = _mlp_call(
      U128, V128, P1u, P1v, b1.reshape(1, EMB_K),
      W2.T, b2.reshape(1, EMB_K),
      W3.reshape(1, EMB_K), b3,
  )
  return out


# speedup vs baseline: 1.6843x; 1.1919x over previous
import functools

import jax
import jax.numpy as jnp
from jax import lax
from jax.experimental import pallas as pl
from jax.experimental.pallas import tpu as pltpu
from jax.experimental.pallas import tpu_sc as plsc

BATCH = 16384
EMB_K = 64
PACKED = 2 * EMB_K
CHUNK = 128
PACK_BLK = 8192
HALF_M = 524288


def _pack_body(wt_lo_ref, wt_hi_ref, ht_lo_ref, ht_hi_ref, eye_ref, p_ref):
  dims = (((0,), (0,)), ((), ()))
  eye = eye_ref[...]

  def pack(wt_ref, ht_ref):
    w = jax.lax.dot_general(wt_ref[...], eye, dims,
                            preferred_element_type=jnp.float32)
    h = jax.lax.dot_general(ht_ref[...], eye, dims,
                            preferred_element_type=jnp.float32)
    half = jnp.int32(0x8000)
    wi = lax.bitcast_convert_type(w, jnp.int32) + half
    hi_ = lax.bitcast_convert_type(h, jnp.int32) + half
    wtop = jnp.bitwise_and(wi, jnp.int32(-65536))
    hbot = lax.shift_right_logical(hi_, 16)
    return jnp.bitwise_or(wtop, hbot)

  p_ref[...] = jnp.concatenate(
      [pack(wt_lo_ref, ht_lo_ref), pack(wt_hi_ref, ht_hi_ref)], axis=1)


def _pack_call(WT, HT):
  eye = jnp.eye(EMB_K, dtype=jnp.float32)
  nblk = HALF_M // PACK_BLK
  last = pl.cdiv(WT.shape[1], PACK_BLK) - 1
  lo = pl.BlockSpec((EMB_K, PACK_BLK), lambda i: (0, i))
  hi = pl.BlockSpec((EMB_K, PACK_BLK),
                    lambda i: (0, jnp.minimum(i + nblk, last)))
  return pl.pallas_call(
      _pack_body,
      grid=(nblk,),
      in_specs=[lo, hi, lo, hi, pl.BlockSpec((EMB_K, EMB_K), lambda i: (0, 0))],
      out_specs=pl.BlockSpec((PACK_BLK, PACKED), lambda i: (i, 0)),
      out_shape=jax.ShapeDtypeStruct((HALF_M, PACKED), jnp.int32),
  )(WT, WT, HT, HT, eye)


def _gather_call(uidx3d, vidx3d, P2):
  info = plsc.get_sparse_core_info()
  nc, ns = info.num_cores, info.num_subcores
  nw = nc * ns
  rows_per_w = BATCH // nw
  chunks_per_w = rows_per_w // CHUNK

  mesh = plsc.VectorSubcoreMesh(core_axis_name="c", subcore_axis_name="s")

  @functools.partial(
      pl.kernel,
      mesh=mesh,
      out_type=[
          jax.ShapeDtypeStruct((BATCH, PACKED), jnp.int32),
          jax.ShapeDtypeStruct((BATCH, PACKED), jnp.int32),
      ],
      scratch_types=[
          pltpu.VMEM((chunks_per_w, 1, CHUNK), jnp.int32),
          pltpu.VMEM((chunks_per_w, 1, CHUNK), jnp.int32),
          pltpu.VMEM((rows_per_w, PACKED), jnp.int32),
          pltpu.SemaphoreType.DMA,
      ],
  )
  def gather_k(uidx_hbm, vidx_hbm, p_hbm, u_out, v_out,
               uidx_v, vidx_v, rows, sem):
    wid = lax.axis_index("s") * nc + lax.axis_index("c")
    idx_base = wid * chunks_per_w
    row_base = wid * rows_per_w
    pltpu.sync_copy(uidx_hbm.at[pl.ds(idx_base, chunks_per_w)], uidx_v)
    pltpu.sync_copy(vidx_hbm.at[pl.ds(idx_base, chunks_per_w)], vidx_v)
    for idx_v, out in ((uidx_v, u_out), (vidx_v, v_out)):
      cps = []
      for c in range(chunks_per_w):
        cps.append(pltpu.async_copy(
            p_hbm.at[idx_v.at[c, 0]], rows.at[pl.ds(c * CHUNK, CHUNK)], sem))
      for cp in cps:
        cp.wait()
      pltpu.sync_copy(rows, out.at[pl.ds(row_base, rows_per_w)])

  return gather_k(uidx3d, vidx3d, P2)


def _unpack(sel_i32, shift_left):
  if shift_left:
    bits = lax.shift_left(sel_i32, 16)
  else:
    bits = jnp.bitwise_and(sel_i32, jnp.int32(-65536))
  return lax.bitcast_convert_type(bits, jnp.float32)


def _mlp_body(u_ref, v_ref, ui_ref, vi_ref, w1a_ref, w1b_ref, b1_ref,
              w2_ref, b2_ref, w3_ref, b3_ref, out_ref):
  um = ui_ref[...] < HALF_M
  vm = vi_ref[...] < HALF_M
  xu = u_ref[...]
  xv = v_ref[...]
  usel = jnp.where(um, xu[:, :EMB_K], xu[:, EMB_K:])
  vsel = jnp.where(vm, xv[:, :EMB_K], xv[:, EMB_K:])
  u = _unpack(usel, False)
  v = _unpack(vsel, True)
  h = jnp.dot(u, w1a_ref[...], preferred_element_type=jnp.float32)
  h += jnp.dot(v, w1b_ref[...], preferred_element_type=jnp.float32)
  h = jnp.maximum(h + b1_ref[...], 0.0)
  h = jnp.dot(h, w2_ref[...], preferred_element_type=jnp.float32)
  h = jnp.maximum(h + b2_ref[...], 0.0)
  out_ref[...] = jnp.sum(h * w3_ref[...], axis=1) + b3_ref[0]


def _mlp_call(U128, V128, uidx, vidx, W1aT, W1bT, b1, W2T, b2, w3, b3):
  blk = 2048
  grid = (BATCH // blk,)
  full = lambda shape: pl.BlockSpec(shape, lambda i: (0,) * len(shape))
  return pl.pallas_call(
      _mlp_body,
      grid=grid,
      in_specs=[
          pl.BlockSpec((blk, PACKED), lambda i: (i, 0)),
          pl.BlockSpec((blk, PACKED), lambda i: (i, 0)),
          pl.BlockSpec((blk, 1), lambda i: (i, 0)),
          pl.BlockSpec((blk, 1), lambda i: (i, 0)),
          full((EMB_K, EMB_K)),
          full((EMB_K, EMB_K)),
          full((1, EMB_K)),
          full((EMB_K, EMB_K)),
          full((1, EMB_K)),
          full((1, EMB_K)),
          full((1,)),
      ],
      out_specs=pl.BlockSpec((blk,), lambda i: (i,)),
      out_shape=jax.ShapeDtypeStruct((BATCH,), jnp.float32),
  )(U128, V128, uidx, vidx, W1aT, W1bT, b1, W2T, b2, w3, b3)


@jax.jit
def kernel(x, W, H, W1, b1, W2, b2, W3, b3):
  uidx = x[:, 0].astype(jnp.int32)
  vidx = x[:, 1].astype(jnp.int32)
  uidx3d = (uidx % HALF_M).reshape(BATCH // CHUNK, 1, CHUNK)
  vidx3d = (vidx % HALF_M).reshape(BATCH // CHUNK, 1, CHUNK)
  P2 = _pack_call(W.T, H.T)
  U128, V128 = _gather_call(uidx3d, vidx3d, P2)
  out = _mlp_call(
      U128, V128, uidx.reshape(BATCH, 1), vidx.reshape(BATCH, 1),
      W1[:, :EMB_K].T, W1[:, EMB_K:].T, b1.reshape(1, EMB_K),
      W2.T, b2.reshape(1, EMB_K),
      W3.reshape(1, EMB_K), b3,
  )
  return out


# speedup vs baseline: 1.6937x; 1.0056x over previous
import functools

import jax
import jax.numpy as jnp
from jax import lax
from jax.experimental import pallas as pl
from jax.experimental.pallas import tpu as pltpu
from jax.experimental.pallas import tpu_sc as plsc

BATCH = 16384
EMB_K = 64
PACKED = 2 * EMB_K
CHUNK = 128
PACK_BLK = 16384
HALF_M = 524288


def _pack_body(wt_lo_ref, wt_hi_ref, ht_lo_ref, ht_hi_ref, eye_ref, p_ref):
  dims = (((0,), (0,)), ((), ()))
  eye = eye_ref[...]

  def pack(wt_ref, ht_ref):
    w = jax.lax.dot_general(wt_ref[...], eye, dims,
                            preferred_element_type=jnp.float32)
    h = jax.lax.dot_general(ht_ref[...], eye, dims,
                            preferred_element_type=jnp.float32)
    half = jnp.int32(0x8000)
    wi = lax.bitcast_convert_type(w, jnp.int32) + half
    hi_ = lax.bitcast_convert_type(h, jnp.int32) + half
    wtop = jnp.bitwise_and(wi, jnp.int32(-65536))
    hbot = lax.shift_right_logical(hi_, 16)
    return jnp.bitwise_or(wtop, hbot)

  p_ref[...] = jnp.concatenate(
      [pack(wt_lo_ref, ht_lo_ref), pack(wt_hi_ref, ht_hi_ref)], axis=1)


def _pack_call(WT, HT):
  eye = jnp.eye(EMB_K, dtype=jnp.float32)
  nblk = HALF_M // PACK_BLK
  last = pl.cdiv(WT.shape[1], PACK_BLK) - 1
  lo = pl.BlockSpec((EMB_K, PACK_BLK), lambda i: (0, i))
  hi = pl.BlockSpec((EMB_K, PACK_BLK),
                    lambda i: (0, jnp.minimum(i + nblk, last)))
  return pl.pallas_call(
      _pack_body,
      grid=(nblk,),
      in_specs=[lo, hi, lo, hi, pl.BlockSpec((EMB_K, EMB_K), lambda i: (0, 0))],
      out_specs=pl.BlockSpec((PACK_BLK, PACKED), lambda i: (i, 0)),
      out_shape=jax.ShapeDtypeStruct((HALF_M, PACKED), jnp.int32),
      compiler_params=pltpu.CompilerParams(vmem_limit_bytes=112 * 1024 * 1024),
  )(WT, WT, HT, HT, eye)


def _gather_call(uidx3d, vidx3d, P2):
  info = plsc.get_sparse_core_info()
  nc, ns = info.num_cores, info.num_subcores
  nw = nc * ns
  rows_per_w = BATCH // nw
  chunks_per_w = rows_per_w // CHUNK

  mesh = plsc.VectorSubcoreMesh(core_axis_name="c", subcore_axis_name="s")

  @functools.partial(
      pl.kernel,
      mesh=mesh,
      out_type=[
          jax.ShapeDtypeStruct((BATCH, PACKED), jnp.int32),
          jax.ShapeDtypeStruct((BATCH, PACKED), jnp.int32),
      ],
      scratch_types=[
          pltpu.VMEM((chunks_per_w, 1, CHUNK), jnp.int32),
          pltpu.VMEM((chunks_per_w, 1, CHUNK), jnp.int32),
          pltpu.VMEM((rows_per_w, PACKED), jnp.int32),
          pltpu.SemaphoreType.DMA,
      ],
  )
  def gather_k(uidx_hbm, vidx_hbm, p_hbm, u_out, v_out,
               uidx_v, vidx_v, rows, sem):
    wid = lax.axis_index("s") * nc + lax.axis_index("c")
    idx_base = wid * chunks_per_w
    row_base = wid * rows_per_w
    pltpu.sync_copy(uidx_hbm.at[pl.ds(idx_base, chunks_per_w)], uidx_v)
    pltpu.sync_copy(vidx_hbm.at[pl.ds(idx_base, chunks_per_w)], vidx_v)
    for idx_v, out in ((uidx_v, u_out), (vidx_v, v_out)):
      cps = []
      for c in range(chunks_per_w):
        cps.append(pltpu.async_copy(
            p_hbm.at[idx_v.at[c, 0]], rows.at[pl.ds(c * CHUNK, CHUNK)], sem))
      for cp in cps:
        cp.wait()
      pltpu.sync_copy(rows, out.at[pl.ds(row_base, rows_per_w)])

  return gather_k(uidx3d, vidx3d, P2)


def _unpack(sel_i32, shift_left):
  if shift_left:
    bits = lax.shift_left(sel_i32, 16)
  else:
    bits = jnp.bitwise_and(sel_i32, jnp.int32(-65536))
  return lax.bitcast_convert_type(bits, jnp.float32)


def _mlp_body(u_ref, v_ref, ui_ref, vi_ref, w1a_ref, w1b_ref, b1_ref,
              w2_ref, b2_ref, w3_ref, b3_ref, out_ref):
  um = ui_ref[...] < HALF_M
  vm = vi_ref[...] < HALF_M
  xu = u_ref[...]
  xv = v_ref[...]
  usel = jnp.where(um, xu[:, :EMB_K], xu[:, EMB_K:])
  vsel = jnp.where(vm, xv[:, :EMB_K], xv[:, EMB_K:])
  u = _unpack(usel, False)
  v = _unpack(vsel, True)
  h = jnp.dot(u, w1a_ref[...], preferred_element_type=jnp.float32)
  h += jnp.dot(v, w1b_ref[...], preferred_element_type=jnp.float32)
  h = jnp.maximum(h + b1_ref[...], 0.0)
  h = jnp.dot(h, w2_ref[...], preferred_element_type=jnp.float32)
  h = jnp.maximum(h + b2_ref[...], 0.0)
  out_ref[...] = jnp.sum(h * w3_ref[...], axis=1) + b3_ref[0]


def _mlp_call(U128, V128, uidx, vidx, W1aT, W1bT, b1, W2T, b2, w3, b3):
  blk = 2048
  grid = (BATCH // blk,)
  full = lambda shape: pl.BlockSpec(shape, lambda i: (0,) * len(shape))
  return pl.pallas_call(
      _mlp_body,
      grid=grid,
      in_specs=[
          pl.BlockSpec((blk, PACKED), lambda i: (i, 0)),
          pl.BlockSpec((blk, PACKED), lambda i: (i, 0)),
          pl.BlockSpec((blk, 1), lambda i: (i, 0)),
          pl.BlockSpec((blk, 1), lambda i: (i, 0)),
          full((EMB_K, EMB_K)),
          full((EMB_K, EMB_K)),
          full((1, EMB_K)),
          full((EMB_K, EMB_K)),
          full((1, EMB_K)),
          full((1, EMB_K)),
          full((1,)),
      ],
      out_specs=pl.BlockSpec((blk,), lambda i: (i,)),
      out_shape=jax.ShapeDtypeStruct((BATCH,), jnp.float32),
  )(U128, V128, uidx, vidx, W1aT, W1bT, b1, W2T, b2, w3, b3)


@jax.jit
def kernel(x, W, H, W1, b1, W2, b2, W3, b3):
  uidx = x[:, 0].astype(jnp.int32)
  vidx = x[:, 1].astype(jnp.int32)
  uidx3d = (uidx % HALF_M).reshape(BATCH // CHUNK, 1, CHUNK)
  vidx3d = (vidx % HALF_M).reshape(BATCH // CHUNK, 1, CHUNK)
  P2 = _pack_call(W.T, H.T)
  U128, V128 = _gather_call(uidx3d, vidx3d, P2)
  out = _mlp_call(
      U128, V128, uidx.reshape(BATCH, 1), vidx.reshape(BATCH, 1),
      W1[:, :EMB_K].T, W1[:, EMB_K:].T, b1.reshape(1, EMB_K),
      W2.T, b2.reshape(1, EMB_K),
      W3.reshape(1, EMB_K), b3,
  )
  return out
